# baseline (speedup 1.0000x reference)
import functools

import jax
import jax.numpy as jnp
from jax import lax
from jax.experimental import pallas as pl
from jax.experimental.pallas import tpu as pltpu
from jax.experimental.pallas import tpu_sc as plsc

H = 1024
FF = 2816
E = 8
T = 2048
TM = 512
NT = (2 * T) // TM + E
NPAD = NT * TM

NC = 2
NS = 16
NW = NC * NS
CB = T // NW
CD = CB // 2


def _route_body(x_ref, wg_ref, pos0_ref, pos1_ref, w0_ref, w1_ref,
                te_ref, rend_ref):
    x = x_ref[...]
    wg = wg_ref[...]
    logits = lax.dot_general(x, wg, (((1,), (1,)), ((), ())),
                             preferred_element_type=jnp.float32)
    iota_e = lax.broadcasted_iota(jnp.int32, (T, E), 1)
    m0 = jnp.max(logits, axis=1, keepdims=True)
    i0 = jnp.min(jnp.where(logits == m0, iota_e, E), axis=1, keepdims=True)
    oh0 = iota_e == i0
    masked = jnp.where(oh0, -1e30, logits)
    m1 = jnp.max(masked, axis=1, keepdims=True)
    i1 = jnp.min(jnp.where(masked == m1, iota_e, E), axis=1, keepdims=True)
    oh1 = iota_e == i1
    w0 = 1.0 / (1.0 + jnp.exp(m1 - m0))
    w0_ref[...] = w0
    w1_ref[...] = 1.0 - w0

    ohs = oh0.astype(jnp.float32) + oh1.astype(jnp.float32)
    s = ohs
    d = 1
    while d < T:
        shifted = jnp.concatenate(
            [jnp.zeros((d, E), jnp.float32), s[: T - d, :]], axis=0)
        s = s + shifted
        d *= 2
    s_exc = s - ohs
    counts = jnp.sum(ohs, axis=0, keepdims=True)
    pc = jnp.ceil(counts / TM) * TM
    ii = lax.broadcasted_iota(jnp.int32, (E, E), 0)
    jj = lax.broadcasted_iota(jnp.int32, (E, E), 1)
    mstrict = (ii < jj).astype(jnp.float32)
    po = lax.dot_general(pc, mstrict, (((1,), (0,)), ((), ())),
                         preferred_element_type=jnp.float32)
    oh0f = oh0.astype(jnp.float32)
    oh1f = oh1.astype(jnp.float32)
    pos0 = jnp.sum(s_exc * oh0f + po * oh0f, axis=1, keepdims=True)
    pos1 = jnp.sum(s_exc * oh1f + po * oh1f, axis=1, keepdims=True)
    pos0_ref[...] = pos0.astype(jnp.int32)
    pos1_ref[...] = pos1.astype(jnp.int32)

    ends_pad = po + pc
    ends_real = po + counts
    ts = (lax.broadcasted_iota(jnp.int32, (NT, E), 0) * TM).astype(jnp.float32)
    te = jnp.sum((ts >= ends_pad).astype(jnp.int32), axis=1, keepdims=True)
    te = jnp.minimum(te, E - 1)
    ohte = (lax.broadcasted_iota(jnp.int32, (NT, E), 1) == te)
    rend = jnp.sum(ohte.astype(jnp.float32) * ends_real, axis=1, keepdims=True)
    te_ref[...] = te
    rend_ref[...] = rend.astype(jnp.int32)


def _route(h, wg):
    f32 = jnp.float32
    i32 = jnp.int32
    return pl.pallas_call(
        _route_body,
        out_shape=[
            jax.ShapeDtypeStruct((T, 1), i32),
            jax.ShapeDtypeStruct((T, 1), i32),
            jax.ShapeDtypeStruct((T, 1), f32),
            jax.ShapeDtypeStruct((T, 1), f32),
            jax.ShapeDtypeStruct((NT, 1), i32),
            jax.ShapeDtypeStruct((NT, 1), i32),
        ],
    )(h, wg)


@functools.cache
def _sc_kernels():
    mesh = plsc.VectorSubcoreMesh(core_axis_name="c", subcore_axis_name="s")

    @functools.partial(
        pl.kernel,
        mesh=mesh,
        out_type=jax.ShapeDtypeStruct((NPAD, H), jnp.float32),
        scratch_types=[
            pltpu.VMEM((CB, H), jnp.float32),
            pltpu.VMEM((CB,), jnp.int32),
            pltpu.VMEM((CB,), jnp.int32),
            pltpu.SemaphoreType.DMA,
            pltpu.SemaphoreType.DMA,
        ],
    )
    def _sc_scatter(x_hbm, pos0_hbm, pos1_hbm, xs_hbm, xbuf, i0buf, i1buf,
                    sem0, sem1):
        wid = lax.axis_index("s") * NC + lax.axis_index("c")
        base = wid * CB
        pltpu.sync_copy(x_hbm.at[pl.ds(base, CB)], xbuf)
        pltpu.sync_copy(pos0_hbm.at[pl.ds(base, CB)], i0buf)
        pltpu.sync_copy(pos1_hbm.at[pl.ds(base, CB)], i1buf)
        c0 = pltpu.async_copy(xbuf, xs_hbm.at[i0buf], sem0)
        c1 = pltpu.async_copy(xbuf, xs_hbm.at[i1buf], sem1)
        c0.wait()
        c1.wait()

    @functools.partial(
        pl.kernel,
        mesh=mesh,
        out_type=(
            jax.ShapeDtypeStruct((T, H), jnp.float32),
            jax.ShapeDtypeStruct((T, H), jnp.float32),
        ),
        scratch_types=[
            pltpu.VMEM((CD, H), jnp.float32),
            pltpu.VMEM((CD, H), jnp.float32),
            pltpu.VMEM((CD,), jnp.int32),
            pltpu.VMEM((CD,), jnp.int32),
            pltpu.SemaphoreType.DMA,
            pltpu.SemaphoreType.DMA,
        ],
    )
    def _sc_gather(outs_hbm, pos0_hbm, pos1_hbm, r0_hbm, r1_hbm,
                   b0, b1, i0buf, i1buf, sem0, sem1):
        wid = lax.axis_index("s") * NC + lax.axis_index("c")
        for half in range(CB // CD):
            base = wid * CB + half * CD
            pltpu.sync_copy(pos0_hbm.at[pl.ds(base, CD)], i0buf)
            pltpu.sync_copy(pos1_hbm.at[pl.ds(base, CD)], i1buf)
            c0 = pltpu.async_copy(outs_hbm.at[i0buf], b0, sem0)
            c1 = pltpu.async_copy(outs_hbm.at[i1buf], b1, sem1)
            c0.wait()
            c1.wait()
            pltpu.sync_copy(b0, r0_hbm.at[pl.ds(base, CD)])
            pltpu.sync_copy(b1, r1_hbm.at[pl.ds(base, CD)])

    return _sc_scatter, _sc_gather


def _ffn_body(te_ref, rend_ref, xs_ref, w1_ref, w2_ref, out_ref):
    sidx = pl.program_id(0)
    end = rend_ref[sidx]

    @pl.when(end > sidx * TM)
    def _():
        rows = sidx * TM + lax.broadcasted_iota(jnp.int32, (TM, 1), 0)
        xv = jnp.where(rows < end, xs_ref[...], 0.0)
        hmid = lax.dot_general(xv, w1_ref[0], (((1,), (1,)), ((), ())),
                               preferred_element_type=jnp.float32)
        hmid = hmid * lax.logistic(hmid)
        out_ref[...] = lax.dot_general(
            hmid, w2_ref[0], (((1,), (1,)), ((), ())),
            preferred_element_type=jnp.float32)


def _grouped_ffn(xs, w1, w2, te, rend):
    grid_spec = pltpu.PrefetchScalarGridSpec(
        num_scalar_prefetch=2,
        grid=(NT,),
        in_specs=[
            pl.BlockSpec((TM, H), lambda s, te_r, re_r: (s, 0)),
            pl.BlockSpec((1, FF, H), lambda s, te_r, re_r: (te_r[s], 0, 0)),
            pl.BlockSpec((1, H, FF), lambda s, te_r, re_r: (te_r[s], 0, 0)),
        ],
        out_specs=pl.BlockSpec((TM, H), lambda s, te_r, re_r: (s, 0)),
    )
    return pl.pallas_call(
        _ffn_body,
        grid_spec=grid_spec,
        out_shape=jax.ShapeDtypeStruct((NPAD, H), jnp.float32),
    )(te, rend, xs, w1, w2)


def _combine_body(w0_ref, w1_ref, r0_ref, r1_ref, y_ref):
    y_ref[...] = w0_ref[...] * r0_ref[...] + w1_ref[...] * r1_ref[...]


def _combine(w0, w1, r0, r1):
    cb = 256
    return pl.pallas_call(
        _combine_body,
        grid=(T // cb,),
        in_specs=[
            pl.BlockSpec((cb, 1), lambda i: (i, 0)),
            pl.BlockSpec((cb, 1), lambda i: (i, 0)),
            pl.BlockSpec((cb, H), lambda i: (i, 0)),
            pl.BlockSpec((cb, H), lambda i: (i, 0)),
        ],
        out_specs=pl.BlockSpec((cb, H), lambda i: (i, 0)),
        out_shape=jax.ShapeDtypeStruct((T, H), jnp.float32),
    )(w0, w1, r0, r1)


def kernel(x, Wg, W1, W2):
    b, t, d = x.shape
    assert (b * t, d) == (T, H) and W1.shape == (E, FF, H)
    h = x.reshape(T, H)
    pos0, pos1, w0, w1, te, rend = _route(h, Wg)
    p0 = pos0.reshape(T)
    p1 = pos1.reshape(T)
    sc_scatter, sc_gather = _sc_kernels()
    xs = sc_scatter(h, p0, p1)
    outs = _grouped_ffn(xs, W1, W2, te.reshape(NT), rend.reshape(NT))
    r0, r1 = sc_gather(outs, p0, p1)
    y = _combine(w0, w1, r0, r1)
    return y.reshape(b, t, d)


# speedup vs baseline: 1.0456x; 1.0456x over previous
import functools

import jax
import jax.numpy as jnp
from jax import lax
from jax.experimental import pallas as pl
from jax.experimental.pallas import tpu as pltpu
from jax.experimental.pallas import tpu_sc as plsc

H = 1024
FF = 2816
E = 8
T = 2048
TM = 512
NT = (2 * T) // TM + E
NPAD = NT * TM

NC = 2
NS = 16
NW = NC * NS
CB = T // NW
CG = 16


def _route_body(x_ref, wg_ref, pos0_ref, pos1_ref, w0_ref, w1_ref,
                te_ref, rend_ref):
    x = x_ref[...]
    wg = wg_ref[...]
    logits = lax.dot_general(x, wg, (((1,), (1,)), ((), ())),
                             preferred_element_type=jnp.float32)
    iota_e = lax.broadcasted_iota(jnp.int32, (T, E), 1)
    m0 = jnp.max(logits, axis=1, keepdims=True)
    i0 = jnp.min(jnp.where(logits == m0, iota_e, E), axis=1, keepdims=True)
    oh0 = iota_e == i0
    masked = jnp.where(oh0, -1e30, logits)
    m1 = jnp.max(masked, axis=1, keepdims=True)
    i1 = jnp.min(jnp.where(masked == m1, iota_e, E), axis=1, keepdims=True)
    oh1 = iota_e == i1
    w0 = 1.0 / (1.0 + jnp.exp(m1 - m0))
    w0_ref[...] = jnp.broadcast_to(w0, (T, 16))
    w1_ref[...] = jnp.broadcast_to(1.0 - w0, (T, 16))

    ohs = oh0.astype(jnp.float32) + oh1.astype(jnp.float32)
    s = ohs
    d = 1
    while d < T:
        shifted = jnp.concatenate(
            [jnp.zeros((d, E), jnp.float32), s[: T - d, :]], axis=0)
        s = s + shifted
        d *= 2
    s_exc = s - ohs
    counts = jnp.sum(ohs, axis=0, keepdims=True)
    pc = jnp.ceil(counts / TM) * TM
    ii = lax.broadcasted_iota(jnp.int32, (E, E), 0)
    jj = lax.broadcasted_iota(jnp.int32, (E, E), 1)
    mstrict = (ii < jj).astype(jnp.float32)
    po = lax.dot_general(pc, mstrict, (((1,), (0,)), ((), ())),
                         preferred_element_type=jnp.float32)
    oh0f = oh0.astype(jnp.float32)
    oh1f = oh1.astype(jnp.float32)
    pos0 = jnp.sum(s_exc * oh0f + po * oh0f, axis=1, keepdims=True)
    pos1 = jnp.sum(s_exc * oh1f + po * oh1f, axis=1, keepdims=True)
    pos0_ref[...] = pos0.astype(jnp.int32)
    pos1_ref[...] = pos1.astype(jnp.int32)

    ends_pad = po + pc
    ends_real = po + counts
    ts = (lax.broadcasted_iota(jnp.int32, (NT, E), 0) * TM).astype(jnp.float32)
    te = jnp.sum((ts >= ends_pad).astype(jnp.int32), axis=1, keepdims=True)
    te = jnp.minimum(te, E - 1)
    ohte = (lax.broadcasted_iota(jnp.int32, (NT, E), 1) == te)
    rend = jnp.sum(ohte.astype(jnp.float32) * ends_real, axis=1, keepdims=True)
    te_ref[...] = te
    rend_ref[...] = rend.astype(jnp.int32)


def _route(h, wg):
    f32 = jnp.float32
    i32 = jnp.int32
    return pl.pallas_call(
        _route_body,
        out_shape=[
            jax.ShapeDtypeStruct((T, 1), i32),
            jax.ShapeDtypeStruct((T, 1), i32),
            jax.ShapeDtypeStruct((T, 16), f32),
            jax.ShapeDtypeStruct((T, 16), f32),
            jax.ShapeDtypeStruct((NT, 1), i32),
            jax.ShapeDtypeStruct((NT, 1), i32),
        ],
    )(h, wg)


@functools.cache
def _sc_kernels():
    mesh = plsc.VectorSubcoreMesh(core_axis_name="c", subcore_axis_name="s")

    @functools.partial(
        pl.kernel,
        mesh=mesh,
        out_type=jax.ShapeDtypeStruct((NPAD, H), jnp.float32),
        scratch_types=[
            pltpu.VMEM((CB, H), jnp.float32),
            pltpu.VMEM((CB,), jnp.int32),
            pltpu.VMEM((CB,), jnp.int32),
            pltpu.SemaphoreType.DMA,
            pltpu.SemaphoreType.DMA,
        ],
    )
    def _sc_scatter(x_hbm, pos0_hbm, pos1_hbm, xs_hbm, xbuf, i0buf, i1buf,
                    sem0, sem1):
        wid = lax.axis_index("s") * NC + lax.axis_index("c")
        base = wid * CB
        pltpu.sync_copy(x_hbm.at[pl.ds(base, CB)], xbuf)
        pltpu.sync_copy(pos0_hbm.at[pl.ds(base, CB)], i0buf)
        pltpu.sync_copy(pos1_hbm.at[pl.ds(base, CB)], i1buf)
        c0 = pltpu.async_copy(xbuf, xs_hbm.at[i0buf], sem0)
        c1 = pltpu.async_copy(xbuf, xs_hbm.at[i1buf], sem1)
        c0.wait()
        c1.wait()

    @functools.partial(
        pl.kernel,
        mesh=mesh,
        out_type=jax.ShapeDtypeStruct((T, H), jnp.float32),
        scratch_types=[
            pltpu.VMEM((CG, H), jnp.float32),
            pltpu.VMEM((CG, H), jnp.float32),
            pltpu.VMEM((CG, H), jnp.float32),
            pltpu.VMEM((CG, H), jnp.float32),
            pltpu.VMEM((CG, H), jnp.float32),
            pltpu.VMEM((CG, H), jnp.float32),
            pltpu.VMEM((CB, 16), jnp.float32),
            pltpu.VMEM((CB, 16), jnp.float32),
            pltpu.VMEM((CB,), jnp.int32),
            pltpu.VMEM((CB,), jnp.int32),
            pltpu.SemaphoreType.DMA,
            pltpu.SemaphoreType.DMA,
            pltpu.SemaphoreType.DMA,
            pltpu.SemaphoreType.DMA,
            pltpu.SemaphoreType.DMA,
            pltpu.SemaphoreType.DMA,
        ],
    )
    def _sc_combine(outs_hbm, pos0_hbm, pos1_hbm, w0_hbm, w1_hbm, y_hbm,
                    g0a, g1a, g0b, g1b, yba, ybb, w0b_, w1b_, i0buf, i1buf,
                    sg0a, sg1a, sg0b, sg1b, sya, syb):
        wid = lax.axis_index("s") * NC + lax.axis_index("c")
        wbase = wid * CB
        pltpu.sync_copy(pos0_hbm.at[pl.ds(wbase, CB)], i0buf)
        pltpu.sync_copy(pos1_hbm.at[pl.ds(wbase, CB)], i1buf)
        pltpu.sync_copy(w0_hbm.at[pl.ds(wbase, CB)], w0b_)
        pltpu.sync_copy(w1_hbm.at[pl.ds(wbase, CB)], w1b_)
        g0 = (g0a, g0b)
        g1 = (g1a, g1b)
        yb = (yba, ybb)
        sg0 = (sg0a, sg0b)
        sg1 = (sg1a, sg1b)
        sy = (sya, syb)
        nch = CB // CG
        gathers = [None] * nch
        writes = [None] * nch

        def fire(c):
            p = c % 2
            sl = pl.ds(c * CG, CG)
            gathers[c] = (
                pltpu.async_copy(outs_hbm.at[i0buf.at[sl]], g0[p], sg0[p]),
                pltpu.async_copy(outs_hbm.at[i1buf.at[sl]], g1[p], sg1[p]),
            )

        fire(0)
        for c in range(nch):
            p = c % 2
            if c + 1 < nch:
                fire(c + 1)
            ca, cb = gathers[c]
            ca.wait()
            cb.wait()
            if c >= 2:
                writes[c - 2].wait()

            def _row(i, _):
                w0v = w0b_[c * CG + i]
                w1v = w1b_[c * CG + i]
                for j in range(H // 16):
                    sl = pl.ds(j * 16, 16)
                    yb[p][i, sl] = w0v * g0[p][i, sl] + w1v * g1[p][i, sl]
                return _

            lax.fori_loop(0, CG, _row, 0)
            writes[c] = pltpu.async_copy(
                yb[p], y_hbm.at[pl.ds(wbase + c * CG, CG)], sy[p])
        writes[nch - 2].wait()
        writes[nch - 1].wait()

    return _sc_scatter, _sc_combine


def _ffn_body(te_ref, rend_ref, xs_ref, w1_ref, w2_ref, out_ref):
    sidx = pl.program_id(0)
    end = rend_ref[sidx]

    @pl.when(end > sidx * TM)
    def _():
        rows = sidx * TM + lax.broadcasted_iota(jnp.int32, (TM, 1), 0)
        xv = jnp.where(rows < end, xs_ref[...], 0.0)
        hmid = lax.dot_general(xv, w1_ref[0], (((1,), (1,)), ((), ())),
                               preferred_element_type=jnp.float32)
        hmid = hmid * lax.logistic(hmid)
        out_ref[...] = lax.dot_general(
            hmid, w2_ref[0], (((1,), (1,)), ((), ())),
            preferred_element_type=jnp.float32)


def _grouped_ffn(xs, w1, w2, te, rend):
    grid_spec = pltpu.PrefetchScalarGridSpec(
        num_scalar_prefetch=2,
        grid=(NT,),
        in_specs=[
            pl.BlockSpec((TM, H), lambda s, te_r, re_r: (s, 0)),
            pl.BlockSpec((1, FF, H), lambda s, te_r, re_r: (te_r[s], 0, 0)),
            pl.BlockSpec((1, H, FF), lambda s, te_r, re_r: (te_r[s], 0, 0)),
        ],
        out_specs=pl.BlockSpec((TM, H), lambda s, te_r, re_r: (s, 0)),
    )
    return pl.pallas_call(
        _ffn_body,
        grid_spec=grid_spec,
        out_shape=jax.ShapeDtypeStruct((NPAD, H), jnp.float32),
    )(te, rend, xs, w1, w2)


def kernel(x, Wg, W1, W2):
    b, t, d = x.shape
    assert (b * t, d) == (T, H) and W1.shape == (E, FF, H)
    h = x.reshape(T, H)
    pos0, pos1, w0, w1, te, rend = _route(h, Wg)
    p0 = pos0.reshape(T)
    p1 = pos1.reshape(T)
    sc_scatter, sc_combine = _sc_kernels()
    xs = sc_scatter(h, p0, p1)
    outs = _grouped_ffn(xs, W1, W2, te.reshape(NT), rend.reshape(NT))
    y = sc_combine(outs, p0, p1, w0, w1)
    return y.reshape(b, t, d)


# speedup vs baseline: 1.0674x; 1.0208x over previous
import functools

import jax
import jax.numpy as jnp
from jax import lax
from jax.experimental import pallas as pl
from jax.experimental.pallas import tpu as pltpu
from jax.experimental.pallas import tpu_sc as plsc

H = 1024
FF = 2816
E = 8
T = 2048
TM = 512
NT = (2 * T) // TM + E
NPAD = NT * TM

NC = 2
NS = 16
NW = NC * NS
CB = T // NW
CG = 16


def _route_body(x_ref, wg_ref, pos0_ref, pos1_ref, w0_ref, w1_ref,
                te_ref, rend_ref):
    x = x_ref[...]
    wg = wg_ref[...]
    logits = lax.dot_general(x, wg, (((1,), (1,)), ((), ())),
                             preferred_element_type=jnp.float32)
    iota_e = lax.broadcasted_iota(jnp.int32, (T, E), 1)
    m0 = jnp.max(logits, axis=1, keepdims=True)
    i0 = jnp.min(jnp.where(logits == m0, iota_e, E), axis=1, keepdims=True)
    oh0 = iota_e == i0
    masked = jnp.where(oh0, -1e30, logits)
    m1 = jnp.max(masked, axis=1, keepdims=True)
    i1 = jnp.min(jnp.where(masked == m1, iota_e, E), axis=1, keepdims=True)
    oh1 = iota_e == i1
    w0 = 1.0 / (1.0 + jnp.exp(m1 - m0))
    w0_ref[...] = jnp.broadcast_to(w0, (T, 16))
    w1_ref[...] = jnp.broadcast_to(1.0 - w0, (T, 16))

    ohs = oh0.astype(jnp.float32) + oh1.astype(jnp.float32)
    s = ohs
    d = 1
    while d < T:
        shifted = jnp.concatenate(
            [jnp.zeros((d, E), jnp.float32), s[: T - d, :]], axis=0)
        s = s + shifted
        d *= 2
    s_exc = s - ohs
    counts = jnp.sum(ohs, axis=0, keepdims=True)
    pc = jnp.ceil(counts / TM) * TM
    ii = lax.broadcasted_iota(jnp.int32, (E, E), 0)
    jj = lax.broadcasted_iota(jnp.int32, (E, E), 1)
    mstrict = (ii < jj).astype(jnp.float32)
    po = lax.dot_general(pc, mstrict, (((1,), (0,)), ((), ())),
                         preferred_element_type=jnp.float32)
    oh0f = oh0.astype(jnp.float32)
    oh1f = oh1.astype(jnp.float32)
    pos0 = jnp.sum(s_exc * oh0f + po * oh0f, axis=1, keepdims=True)
    pos1 = jnp.sum(s_exc * oh1f + po * oh1f, axis=1, keepdims=True)
    pos0_ref[...] = pos0.astype(jnp.int32)
    pos1_ref[...] = pos1.astype(jnp.int32)

    ends_pad = po + pc
    ends_real = po + counts
    ts = (lax.broadcasted_iota(jnp.int32, (NT, E), 0) * TM).astype(jnp.float32)
    te = jnp.sum((ts >= ends_pad).astype(jnp.int32), axis=1, keepdims=True)
    te = jnp.minimum(te, E - 1)
    ohte = (lax.broadcasted_iota(jnp.int32, (NT, E), 1) == te)
    rend = jnp.sum(ohte.astype(jnp.float32) * ends_real, axis=1, keepdims=True)
    te_ref[...] = te
    rend_ref[...] = rend.astype(jnp.int32)


def _route(h, wg):
    f32 = jnp.float32
    i32 = jnp.int32
    return pl.pallas_call(
        _route_body,
        out_shape=[
            jax.ShapeDtypeStruct((T, 1), i32),
            jax.ShapeDtypeStruct((T, 1), i32),
            jax.ShapeDtypeStruct((T, 16), f32),
            jax.ShapeDtypeStruct((T, 16), f32),
            jax.ShapeDtypeStruct((NT, 1), i32),
            jax.ShapeDtypeStruct((NT, 1), i32),
        ],
    )(h, wg)


@functools.cache
def _sc_kernels():
    mesh = plsc.VectorSubcoreMesh(core_axis_name="c", subcore_axis_name="s")

    @functools.partial(
        pl.kernel,
        mesh=mesh,
        out_type=jax.ShapeDtypeStruct((NPAD, H), jnp.float32),
        scratch_types=[
            pltpu.VMEM((CB, H), jnp.float32),
            pltpu.VMEM((CB,), jnp.int32),
            pltpu.VMEM((CB,), jnp.int32),
            pltpu.SemaphoreType.DMA,
            pltpu.SemaphoreType.DMA,
        ],
    )
    def _sc_scatter(x_hbm, pos0_hbm, pos1_hbm, xs_hbm, xbuf, i0buf, i1buf,
                    sem0, sem1):
        wid = lax.axis_index("s") * NC + lax.axis_index("c")
        base = wid * CB
        pltpu.sync_copy(x_hbm.at[pl.ds(base, CB)], xbuf)
        pltpu.sync_copy(pos0_hbm.at[pl.ds(base, CB)], i0buf)
        pltpu.sync_copy(pos1_hbm.at[pl.ds(base, CB)], i1buf)
        c0 = pltpu.async_copy(xbuf, xs_hbm.at[i0buf], sem0)
        c1 = pltpu.async_copy(xbuf, xs_hbm.at[i1buf], sem1)
        c0.wait()
        c1.wait()

    @functools.partial(
        pl.kernel,
        mesh=mesh,
        out_type=jax.ShapeDtypeStruct((T, H), jnp.float32),
        scratch_types=[
            pltpu.VMEM((CG, H), jnp.float32),
            pltpu.VMEM((CG, H), jnp.float32),
            pltpu.VMEM((CG, H), jnp.float32),
            pltpu.VMEM((CG, H), jnp.float32),
            pltpu.VMEM((CG, H), jnp.float32),
            pltpu.VMEM((CG, H), jnp.float32),
            pltpu.VMEM((CB, 16), jnp.float32),
            pltpu.VMEM((CB, 16), jnp.float32),
            pltpu.VMEM((CB,), jnp.int32),
            pltpu.VMEM((CB,), jnp.int32),
            pltpu.SemaphoreType.DMA,
            pltpu.SemaphoreType.DMA,
            pltpu.SemaphoreType.DMA,
            pltpu.SemaphoreType.DMA,
            pltpu.SemaphoreType.DMA,
            pltpu.SemaphoreType.DMA,
        ],
    )
    def _sc_combine(outs_hbm, pos0_hbm, pos1_hbm, w0_hbm, w1_hbm, y_hbm,
                    g0a, g1a, g0b, g1b, yba, ybb, w0b_, w1b_, i0buf, i1buf,
                    sg0a, sg1a, sg0b, sg1b, sya, syb):
        wid = lax.axis_index("s") * NC + lax.axis_index("c")
        wbase = wid * CB
        pltpu.sync_copy(pos0_hbm.at[pl.ds(wbase, CB)], i0buf)
        pltpu.sync_copy(pos1_hbm.at[pl.ds(wbase, CB)], i1buf)
        pltpu.sync_copy(w0_hbm.at[pl.ds(wbase, CB)], w0b_)
        pltpu.sync_copy(w1_hbm.at[pl.ds(wbase, CB)], w1b_)
        g0 = (g0a, g0b)
        g1 = (g1a, g1b)
        yb = (yba, ybb)
        sg0 = (sg0a, sg0b)
        sg1 = (sg1a, sg1b)
        sy = (sya, syb)
        nch = CB // CG
        gathers = [None] * nch
        writes = [None] * nch

        def fire(c):
            p = c % 2
            sl = pl.ds(c * CG, CG)
            gathers[c] = (
                pltpu.async_copy(outs_hbm.at[i0buf.at[sl]], g0[p], sg0[p]),
                pltpu.async_copy(outs_hbm.at[i1buf.at[sl]], g1[p], sg1[p]),
            )

        fire(0)
        for c in range(nch):
            p = c % 2
            if c + 1 < nch:
                fire(c + 1)
            ca, cb = gathers[c]
            ca.wait()
            cb.wait()
            if c >= 2:
                writes[c - 2].wait()

            def _row(i, _):
                w0v = w0b_[c * CG + i]
                w1v = w1b_[c * CG + i]
                for j in range(H // 16):
                    sl = pl.ds(j * 16, 16)
                    yb[p][i, sl] = w0v * g0[p][i, sl] + w1v * g1[p][i, sl]
                return _

            lax.fori_loop(0, CG, _row, 0)
            writes[c] = pltpu.async_copy(
                yb[p], y_hbm.at[pl.ds(wbase + c * CG, CG)], sy[p])
        writes[nch - 2].wait()
        writes[nch - 1].wait()

    return _sc_scatter, _sc_combine


def _ffn_body(te_ref, rend_ref, xs_ref, w1_hbm, w2_hbm, out_ref,
              w1b, w2b, st_ref, sem1, sem2):
    sidx = pl.program_id(0)
    e = te_ref[sidx]
    end = rend_ref[sidx]

    @pl.when(sidx == 0)
    def _():
        st_ref[0] = -1
        st_ref[1] = -1

    def w_copies(f):
        slot = lax.rem(f, 2)
        return (
            pltpu.make_async_copy(w1_hbm.at[f], w1b.at[slot], sem1.at[slot]),
            pltpu.make_async_copy(w2_hbm.at[f], w2b.at[slot], sem2.at[slot]),
        )

    def drain(upto):
        def cond(w):
            return w < upto

        def body(w):
            c1, c2 = w_copies(w + 1)
            c1.wait()
            c2.wait()
            return w + 1

        st_ref[1] = lax.while_loop(cond, body, st_ref[1])

    def fcond(f):
        return f < jnp.minimum(e + 1, E - 1)

    def fbody(f):
        drain(f - 1)
        c1, c2 = w_copies(f + 1)
        c1.start()
        c2.start()
        return f + 1

    st_ref[0] = lax.while_loop(fcond, fbody, st_ref[0])
    drain(e)

    @pl.when(end > sidx * TM)
    def _():
        slot = lax.rem(e, 2)
        rows = sidx * TM + lax.broadcasted_iota(jnp.int32, (TM, 1), 0)
        xv = jnp.where(rows < end, xs_ref[...], 0.0)
        hmid = lax.dot_general(xv, w1b[slot], (((1,), (1,)), ((), ())),
                               preferred_element_type=jnp.float32)
        hmid = hmid * lax.logistic(hmid)
        out_ref[...] = lax.dot_general(
            hmid, w2b[slot], (((1,), (1,)), ((), ())),
            preferred_element_type=jnp.float32)


def _grouped_ffn(xs, w1, w2, te, rend):
    grid_spec = pltpu.PrefetchScalarGridSpec(
        num_scalar_prefetch=2,
        grid=(NT,),
        in_specs=[
            pl.BlockSpec((TM, H), lambda s, te_r, re_r: (s, 0)),
            pl.BlockSpec(memory_space=pl.ANY),
            pl.BlockSpec(memory_space=pl.ANY),
        ],
        out_specs=pl.BlockSpec((TM, H), lambda s, te_r, re_r: (s, 0)),
        scratch_shapes=[
            pltpu.VMEM((2, FF, H), jnp.float32),
            pltpu.VMEM((2, H, FF), jnp.float32),
            pltpu.SMEM((2,), jnp.int32),
            pltpu.SemaphoreType.DMA((2,)),
            pltpu.SemaphoreType.DMA((2,)),
        ],
    )
    return pl.pallas_call(
        _ffn_body,
        grid_spec=grid_spec,
        out_shape=jax.ShapeDtypeStruct((NPAD, H), jnp.float32),
    )(te, rend, xs, w1, w2)


def kernel(x, Wg, W1, W2):
    b, t, d = x.shape
    assert (b * t, d) == (T, H) and W1.shape == (E, FF, H)
    h = x.reshape(T, H)
    pos0, pos1, w0, w1, te, rend = _route(h, Wg)
    p0 = pos0.reshape(T)
    p1 = pos1.reshape(T)
    sc_scatter, sc_combine = _sc_kernels()
    xs = sc_scatter(h, p0, p1)
    outs = _grouped_ffn(xs, W1, W2, te.reshape(NT), rend.reshape(NT))
    y = sc_combine(outs, p0, p1, w0, w1)
    return y.reshape(b, t, d)


# speedup vs baseline: 1.0905x; 1.0217x over previous
import functools

import jax
import jax.numpy as jnp
from jax import lax
from jax.experimental import pallas as pl
from jax.experimental.pallas import tpu as pltpu
from jax.experimental.pallas import tpu_sc as plsc

H = 1024
FF = 2816
E = 8
T = 2048
TM = 256
NT = (2 * T) // TM + E
NPAD = NT * TM

NC = 2
NS = 16
NW = NC * NS
CB = T // NW
CG = 16


def _route_body(x_ref, wg_ref, pos0_ref, pos1_ref, w0_ref, w1_ref,
                te_ref, rend_ref):
    x = x_ref[...]
    wg = wg_ref[...]
    logits = lax.dot_general(x, wg, (((1,), (1,)), ((), ())),
                             preferred_element_type=jnp.float32)
    iota_e = lax.broadcasted_iota(jnp.int32, (T, E), 1)
    m0 = jnp.max(logits, axis=1, keepdims=True)
    i0 = jnp.min(jnp.where(logits == m0, iota_e, E), axis=1, keepdims=True)
    oh0 = iota_e == i0
    masked = jnp.where(oh0, -1e30, logits)
    m1 = jnp.max(masked, axis=1, keepdims=True)
    i1 = jnp.min(jnp.where(masked == m1, iota_e, E), axis=1, keepdims=True)
    oh1 = iota_e == i1
    w0 = 1.0 / (1.0 + jnp.exp(m1 - m0))
    w0_ref[...] = jnp.broadcast_to(w0, (T, 16))
    w1_ref[...] = jnp.broadcast_to(1.0 - w0, (T, 16))

    ohs = oh0.astype(jnp.float32) + oh1.astype(jnp.float32)
    s = ohs
    d = 1
    while d < T:
        shifted = jnp.concatenate(
            [jnp.zeros((d, E), jnp.float32), s[: T - d, :]], axis=0)
        s = s + shifted
        d *= 2
    s_exc = s - ohs
    counts = jnp.sum(ohs, axis=0, keepdims=True)
    pc = jnp.ceil(counts / TM) * TM
    ii = lax.broadcasted_iota(jnp.int32, (E, E), 0)
    jj = lax.broadcasted_iota(jnp.int32, (E, E), 1)
    mstrict = (ii < jj).astype(jnp.float32)
    po = lax.dot_general(pc, mstrict, (((1,), (0,)), ((), ())),
                         preferred_element_type=jnp.float32)
    oh0f = oh0.astype(jnp.float32)
    oh1f = oh1.astype(jnp.float32)
    pos0 = jnp.sum(s_exc * oh0f + po * oh0f, axis=1, keepdims=True)
    pos1 = jnp.sum(s_exc * oh1f + po * oh1f, axis=1, keepdims=True)
    pos0_ref[...] = pos0.astype(jnp.int32)
    pos1_ref[...] = pos1.astype(jnp.int32)

    ends_pad = po + pc
    ends_real = po + counts
    ts = (lax.broadcasted_iota(jnp.int32, (NT, E), 0) * TM).astype(jnp.float32)
    te = jnp.sum((ts >= ends_pad).astype(jnp.int32), axis=1, keepdims=True)
    te = jnp.minimum(te, E - 1)
    ohte = (lax.broadcasted_iota(jnp.int32, (NT, E), 1) == te)
    rend = jnp.sum(ohte.astype(jnp.float32) * ends_real, axis=1, keepdims=True)
    te_ref[...] = te
    rend_ref[...] = rend.astype(jnp.int32)


def _route(h, wg):
    f32 = jnp.float32
    i32 = jnp.int32
    return pl.pallas_call(
        _route_body,
        out_shape=[
            jax.ShapeDtypeStruct((T, 1), i32),
            jax.ShapeDtypeStruct((T, 1), i32),
            jax.ShapeDtypeStruct((T, 16), f32),
            jax.ShapeDtypeStruct((T, 16), f32),
            jax.ShapeDtypeStruct((NT, 1), i32),
            jax.ShapeDtypeStruct((NT, 1), i32),
        ],
    )(h, wg)


@functools.cache
def _sc_kernels():
    mesh = plsc.VectorSubcoreMesh(core_axis_name="c", subcore_axis_name="s")

    @functools.partial(
        pl.kernel,
        mesh=mesh,
        out_type=jax.ShapeDtypeStruct((NPAD, H), jnp.float32),
        scratch_types=[
            pltpu.VMEM((CB, H), jnp.float32),
            pltpu.VMEM((CB,), jnp.int32),
            pltpu.VMEM((CB,), jnp.int32),
            pltpu.SemaphoreType.DMA,
            pltpu.SemaphoreType.DMA,
        ],
    )
    def _sc_scatter(x_hbm, pos0_hbm, pos1_hbm, xs_hbm, xbuf, i0buf, i1buf,
                    sem0, sem1):
        wid = lax.axis_index("s") * NC + lax.axis_index("c")
        base = wid * CB
        pltpu.sync_copy(x_hbm.at[pl.ds(base, CB)], xbuf)
        pltpu.sync_copy(pos0_hbm.at[pl.ds(base, CB)], i0buf)
        pltpu.sync_copy(pos1_hbm.at[pl.ds(base, CB)], i1buf)
        c0 = pltpu.async_copy(xbuf, xs_hbm.at[i0buf], sem0)
        c1 = pltpu.async_copy(xbuf, xs_hbm.at[i1buf], sem1)
        c0.wait()
        c1.wait()

    @functools.partial(
        pl.kernel,
        mesh=mesh,
        out_type=jax.ShapeDtypeStruct((T, H), jnp.float32),
        scratch_types=[
            pltpu.VMEM((CG, H), jnp.float32),
            pltpu.VMEM((CG, H), jnp.float32),
            pltpu.VMEM((CG, H), jnp.float32),
            pltpu.VMEM((CG, H), jnp.float32),
            pltpu.VMEM((CG, H), jnp.float32),
            pltpu.VMEM((CG, H), jnp.float32),
            pltpu.VMEM((CB, 16), jnp.float32),
            pltpu.VMEM((CB, 16), jnp.float32),
            pltpu.VMEM((CB,), jnp.int32),
            pltpu.VMEM((CB,), jnp.int32),
            pltpu.SemaphoreType.DMA,
            pltpu.SemaphoreType.DMA,
            pltpu.SemaphoreType.DMA,
            pltpu.SemaphoreType.DMA,
            pltpu.SemaphoreType.DMA,
            pltpu.SemaphoreType.DMA,
        ],
    )
    def _sc_combine(outs_hbm, pos0_hbm, pos1_hbm, w0_hbm, w1_hbm, y_hbm,
                    g0a, g1a, g0b, g1b, yba, ybb, w0b_, w1b_, i0buf, i1buf,
                    sg0a, sg1a, sg0b, sg1b, sya, syb):
        wid = lax.axis_index("s") * NC + lax.axis_index("c")
        wbase = wid * CB
        pltpu.sync_copy(pos0_hbm.at[pl.ds(wbase, CB)], i0buf)
        pltpu.sync_copy(pos1_hbm.at[pl.ds(wbase, CB)], i1buf)
        pltpu.sync_copy(w0_hbm.at[pl.ds(wbase, CB)], w0b_)
        pltpu.sync_copy(w1_hbm.at[pl.ds(wbase, CB)], w1b_)
        g0 = (g0a, g0b)
        g1 = (g1a, g1b)
        yb = (yba, ybb)
        sg0 = (sg0a, sg0b)
        sg1 = (sg1a, sg1b)
        sy = (sya, syb)
        nch = CB // CG
        gathers = [None] * nch
        writes = [None] * nch

        def fire(c):
            p = c % 2
            sl = pl.ds(c * CG, CG)
            gathers[c] = (
                pltpu.async_copy(outs_hbm.at[i0buf.at[sl]], g0[p], sg0[p]),
                pltpu.async_copy(outs_hbm.at[i1buf.at[sl]], g1[p], sg1[p]),
            )

        fire(0)
        for c in range(nch):
            p = c % 2
            if c + 1 < nch:
                fire(c + 1)
            ca, cb = gathers[c]
            ca.wait()
            cb.wait()
            if c >= 2:
                writes[c - 2].wait()

            def _row(i, _):
                w0v = w0b_[c * CG + i]
                w1v = w1b_[c * CG + i]
                for j in range(H // 16):
                    sl = pl.ds(j * 16, 16)
                    yb[p][i, sl] = w0v * g0[p][i, sl] + w1v * g1[p][i, sl]
                return _

            lax.fori_loop(0, CG, _row, 0)
            writes[c] = pltpu.async_copy(
                yb[p], y_hbm.at[pl.ds(wbase + c * CG, CG)], sy[p])
        writes[nch - 2].wait()
        writes[nch - 1].wait()

    return _sc_scatter, _sc_combine


def _ffn_body(te_ref, rend_ref, xs_ref, w1_hbm, w2_hbm, out_ref,
              w1b, w2b, st_ref, sem1, sem2):
    sidx = pl.program_id(0)
    e = te_ref[sidx]
    end = rend_ref[sidx]

    @pl.when(sidx == 0)
    def _():
        st_ref[0] = -1
        st_ref[1] = -1

    def w_copies(f):
        slot = lax.rem(f, 2)
        return (
            pltpu.make_async_copy(w1_hbm.at[f], w1b.at[slot], sem1.at[slot]),
            pltpu.make_async_copy(w2_hbm.at[f], w2b.at[slot], sem2.at[slot]),
        )

    def drain(upto):
        def cond(w):
            return w < upto

        def body(w):
            c1, c2 = w_copies(w + 1)
            c1.wait()
            c2.wait()
            return w + 1

        st_ref[1] = lax.while_loop(cond, body, st_ref[1])

    def fcond(f):
        return f < jnp.minimum(e + 1, E - 1)

    def fbody(f):
        drain(f - 1)
        c1, c2 = w_copies(f + 1)
        c1.start()
        c2.start()
        return f + 1

    st_ref[0] = lax.while_loop(fcond, fbody, st_ref[0])
    drain(e)

    @pl.when(end > sidx * TM)
    def _():
        slot = lax.rem(e, 2)
        rows = sidx * TM + lax.broadcasted_iota(jnp.int32, (TM, 1), 0)
        xv = jnp.where(rows < end, xs_ref[...], 0.0)
        hmid = lax.dot_general(xv, w1b[slot], (((1,), (1,)), ((), ())),
                               preferred_element_type=jnp.float32)
        hmid = hmid * lax.logistic(hmid)
        out_ref[...] = lax.dot_general(
            hmid, w2b[slot], (((1,), (1,)), ((), ())),
            preferred_element_type=jnp.float32)


def _grouped_ffn(xs, w1, w2, te, rend):
    grid_spec = pltpu.PrefetchScalarGridSpec(
        num_scalar_prefetch=2,
        grid=(NT,),
        in_specs=[
            pl.BlockSpec((TM, H), lambda s, te_r, re_r: (s, 0)),
            pl.BlockSpec(memory_space=pl.ANY),
            pl.BlockSpec(memory_space=pl.ANY),
        ],
        out_specs=pl.BlockSpec((TM, H), lambda s, te_r, re_r: (s, 0)),
        scratch_shapes=[
            pltpu.VMEM((2, FF, H), jnp.float32),
            pltpu.VMEM((2, H, FF), jnp.float32),
            pltpu.SMEM((2,), jnp.int32),
            pltpu.SemaphoreType.DMA((2,)),
            pltpu.SemaphoreType.DMA((2,)),
        ],
    )
    return pl.pallas_call(
        _ffn_body,
        grid_spec=grid_spec,
        out_shape=jax.ShapeDtypeStruct((NPAD, H), jnp.float32),
    )(te, rend, xs, w1, w2)


def kernel(x, Wg, W1, W2):
    b, t, d = x.shape
    assert (b * t, d) == (T, H) and W1.shape == (E, FF, H)
    h = x.reshape(T, H)
    pos0, pos1, w0, w1, te, rend = _route(h, Wg)
    p0 = pos0.reshape(T)
    p1 = pos1.reshape(T)
    sc_scatter, sc_combine = _sc_kernels()
    xs = sc_scatter(h, p0, p1)
    outs = _grouped_ffn(xs, W1, W2, te.reshape(NT), rend.reshape(NT))
    y = sc_combine(outs, p0, p1, w0, w1)
    return y.reshape(b, t, d)


# speedup vs baseline: 1.0949x; 1.0040x over previous
import functools

import jax
import jax.numpy as jnp
from jax import lax
from jax.experimental import pallas as pl
from jax.experimental.pallas import tpu as pltpu
from jax.experimental.pallas import tpu_sc as plsc

H = 1024
FF = 2816
E = 8
T = 2048
TM = 256
NT = (2 * T) // TM + E
NPAD = NT * TM

NC = 2
NS = 16
NW = NC * NS
CB = T // NW
CG = 16


def _route_body(x_ref, wg_ref, pos0_ref, pos1_ref, w0_ref, w1_ref,
                te_ref, rend_ref):
    x = x_ref[...]
    wg = wg_ref[...]
    logits = lax.dot_general(x, wg, (((1,), (1,)), ((), ())),
                             preferred_element_type=jnp.float32)
    iota_e = lax.broadcasted_iota(jnp.int32, (T, E), 1)
    m0 = jnp.max(logits, axis=1, keepdims=True)
    i0 = jnp.min(jnp.where(logits == m0, iota_e, E), axis=1, keepdims=True)
    oh0 = iota_e == i0
    masked = jnp.where(oh0, -1e30, logits)
    m1 = jnp.max(masked, axis=1, keepdims=True)
    i1 = jnp.min(jnp.where(masked == m1, iota_e, E), axis=1, keepdims=True)
    oh1 = iota_e == i1
    w0 = 1.0 / (1.0 + jnp.exp(m1 - m0))
    w0_ref[...] = jnp.broadcast_to(w0, (T, 128))
    w1_ref[...] = jnp.broadcast_to(1.0 - w0, (T, 128))

    ohs = oh0.astype(jnp.float32) + oh1.astype(jnp.float32)
    s = ohs
    d = 1
    while d < T:
        shifted = jnp.concatenate(
            [jnp.zeros((d, E), jnp.float32), s[: T - d, :]], axis=0)
        s = s + shifted
        d *= 2
    s_exc = s - ohs
    counts = jnp.sum(ohs, axis=0, keepdims=True)
    pc = jnp.ceil(counts / TM) * TM
    ii = lax.broadcasted_iota(jnp.int32, (E, E), 0)
    jj = lax.broadcasted_iota(jnp.int32, (E, E), 1)
    mstrict = (ii < jj).astype(jnp.float32)
    po = lax.dot_general(pc, mstrict, (((1,), (0,)), ((), ())),
                         preferred_element_type=jnp.float32)
    oh0f = oh0.astype(jnp.float32)
    oh1f = oh1.astype(jnp.float32)
    pos0 = jnp.sum(s_exc * oh0f + po * oh0f, axis=1, keepdims=True)
    pos1 = jnp.sum(s_exc * oh1f + po * oh1f, axis=1, keepdims=True)
    pos0_ref[...] = pos0.astype(jnp.int32)
    pos1_ref[...] = pos1.astype(jnp.int32)

    ends_pad = po + pc
    ends_real = po + counts
    ts = (lax.broadcasted_iota(jnp.int32, (NT, E), 0) * TM).astype(jnp.float32)
    te = jnp.sum((ts >= ends_pad).astype(jnp.int32), axis=1, keepdims=True)
    te = jnp.minimum(te, E - 1)
    ohte = (lax.broadcasted_iota(jnp.int32, (NT, E), 1) == te)
    rend = jnp.sum(ohte.astype(jnp.float32) * ends_real, axis=1, keepdims=True)
    te_ref[...] = te
    rend_ref[...] = rend.astype(jnp.int32)


def _route(h, wg):
    f32 = jnp.float32
    i32 = jnp.int32
    return pl.pallas_call(
        _route_body,
        out_shape=[
            jax.ShapeDtypeStruct((T, 1), i32),
            jax.ShapeDtypeStruct((T, 1), i32),
            jax.ShapeDtypeStruct((T, 128), f32),
            jax.ShapeDtypeStruct((T, 128), f32),
            jax.ShapeDtypeStruct((NT, 1), i32),
            jax.ShapeDtypeStruct((NT, 1), i32),
        ],
    )(h, wg)


@functools.cache
def _sc_kernels():
    mesh = plsc.VectorSubcoreMesh(core_axis_name="c", subcore_axis_name="s")

    @functools.partial(
        pl.kernel,
        mesh=mesh,
        out_type=(
            jax.ShapeDtypeStruct((NPAD, H), jnp.float32),
            jax.ShapeDtypeStruct((NPAD, 128), jnp.float32),
        ),
        scratch_types=[
            pltpu.VMEM((CB, H), jnp.float32),
            pltpu.VMEM((CB, 128), jnp.float32),
            pltpu.VMEM((CB, 128), jnp.float32),
            pltpu.VMEM((CB,), jnp.int32),
            pltpu.VMEM((CB,), jnp.int32),
            pltpu.SemaphoreType.DMA,
            pltpu.SemaphoreType.DMA,
            pltpu.SemaphoreType.DMA,
            pltpu.SemaphoreType.DMA,
        ],
    )
    def _sc_scatter(x_hbm, w0_hbm, w1_hbm, pos0_hbm, pos1_hbm,
                    xs_hbm, ws_hbm, xbuf, w0buf, w1buf, i0buf, i1buf,
                    sem0, sem1, sem2, sem3):
        wid = lax.axis_index("s") * NC + lax.axis_index("c")
        base = wid * CB
        pltpu.sync_copy(x_hbm.at[pl.ds(base, CB)], xbuf)
        pltpu.sync_copy(w0_hbm.at[pl.ds(base, CB)], w0buf)
        pltpu.sync_copy(w1_hbm.at[pl.ds(base, CB)], w1buf)
        pltpu.sync_copy(pos0_hbm.at[pl.ds(base, CB)], i0buf)
        pltpu.sync_copy(pos1_hbm.at[pl.ds(base, CB)], i1buf)
        c0 = pltpu.async_copy(xbuf, xs_hbm.at[i0buf], sem0)
        c1 = pltpu.async_copy(xbuf, xs_hbm.at[i1buf], sem1)
        c2 = pltpu.async_copy(w0buf, ws_hbm.at[i0buf], sem2)
        c3 = pltpu.async_copy(w1buf, ws_hbm.at[i1buf], sem3)
        c0.wait()
        c1.wait()
        c2.wait()
        c3.wait()

    @functools.partial(
        pl.kernel,
        mesh=mesh,
        out_type=jax.ShapeDtypeStruct((T, H), jnp.float32),
        scratch_types=[
            pltpu.VMEM((CG, H), jnp.float32),
            pltpu.VMEM((CG, H), jnp.float32),
            pltpu.VMEM((CG, H), jnp.float32),
            pltpu.VMEM((CG, H), jnp.float32),
            pltpu.VMEM((CG, H), jnp.float32),
            pltpu.VMEM((CG, H), jnp.float32),
            pltpu.VMEM((CB,), jnp.int32),
            pltpu.VMEM((CB,), jnp.int32),
            pltpu.SemaphoreType.DMA,
            pltpu.SemaphoreType.DMA,
            pltpu.SemaphoreType.DMA,
            pltpu.SemaphoreType.DMA,
            pltpu.SemaphoreType.DMA,
            pltpu.SemaphoreType.DMA,
        ],
    )
    def _sc_combine(outs_hbm, pos0_hbm, pos1_hbm, y_hbm,
                    g0a, g1a, g0b, g1b, yba, ybb, i0buf, i1buf,
                    sg0a, sg1a, sg0b, sg1b, sya, syb):
        wid = lax.axis_index("s") * NC + lax.axis_index("c")
        wbase = wid * CB
        pltpu.sync_copy(pos0_hbm.at[pl.ds(wbase, CB)], i0buf)
        pltpu.sync_copy(pos1_hbm.at[pl.ds(wbase, CB)], i1buf)
        g0 = (g0a, g0b)
        g1 = (g1a, g1b)
        yb = (yba, ybb)
        sg0 = (sg0a, sg0b)
        sg1 = (sg1a, sg1b)
        sy = (sya, syb)
        nch = CB // CG
        gathers = [None] * nch
        writes = [None] * nch

        def fire(c):
            p = c % 2
            sl = pl.ds(c * CG, CG)
            gathers[c] = (
                pltpu.async_copy(outs_hbm.at[i0buf.at[sl]], g0[p], sg0[p]),
                pltpu.async_copy(outs_hbm.at[i1buf.at[sl]], g1[p], sg1[p]),
            )

        fire(0)
        for c in range(nch):
            p = c % 2
            if c + 1 < nch:
                fire(c + 1)
            ca, cb = gathers[c]
            ca.wait()
            cb.wait()
            if c >= 2:
                writes[c - 2].wait()

            def _row(i, _):
                for j in range(H // 16):
                    sl = pl.ds(j * 16, 16)
                    yb[p][i, sl] = g0[p][i, sl] + g1[p][i, sl]
                return _

            lax.fori_loop(0, CG, _row, 0)
            writes[c] = pltpu.async_copy(
                yb[p], y_hbm.at[pl.ds(wbase + c * CG, CG)], sy[p])
        writes[nch - 2].wait()
        writes[nch - 1].wait()

    return _sc_scatter, _sc_combine


def _ffn_body(te_ref, rend_ref, xs_ref, ws_ref, w1_hbm, w2_hbm, out_ref,
              w1b, w2b, st_ref, sem1, sem2):
    sidx = pl.program_id(0)
    e = te_ref[sidx]
    end = rend_ref[sidx]

    @pl.when(sidx == 0)
    def _():
        st_ref[0] = -1
        st_ref[1] = -1

    def w_copies(f):
        slot = lax.rem(f, 2)
        return (
            pltpu.make_async_copy(w1_hbm.at[f], w1b.at[slot], sem1.at[slot]),
            pltpu.make_async_copy(w2_hbm.at[f], w2b.at[slot], sem2.at[slot]),
        )

    def drain(upto):
        def cond(w):
            return w < upto

        def body(w):
            c1, c2 = w_copies(w + 1)
            c1.wait()
            c2.wait()
            return w + 1

        st_ref[1] = lax.while_loop(cond, body, st_ref[1])

    def fcond(f):
        return f < jnp.minimum(e + 1, E - 1)

    def fbody(f):
        drain(f - 1)
        c1, c2 = w_copies(f + 1)
        c1.start()
        c2.start()
        return f + 1

    st_ref[0] = lax.while_loop(fcond, fbody, st_ref[0])
    drain(e)

    @pl.when(end > sidx * TM)
    def _():
        slot = lax.rem(e, 2)
        rows = sidx * TM + lax.broadcasted_iota(jnp.int32, (TM, 1), 0)
        xv = jnp.where(rows < end, xs_ref[...], 0.0)
        hmid = lax.dot_general(xv, w1b[slot], (((1,), (1,)), ((), ())),
                               preferred_element_type=jnp.float32)
        hmid = hmid * lax.logistic(hmid)
        out = lax.dot_general(hmid, w2b[slot], (((1,), (1,)), ((), ())),
                              preferred_element_type=jnp.float32)
        out_ref[...] = out * ws_ref[:, :1]


def _grouped_ffn(xs, ws, w1, w2, te, rend):
    grid_spec = pltpu.PrefetchScalarGridSpec(
        num_scalar_prefetch=2,
        grid=(NT,),
        in_specs=[
            pl.BlockSpec((TM, H), lambda s, te_r, re_r: (s, 0)),
            pl.BlockSpec((TM, 128), lambda s, te_r, re_r: (s, 0)),
            pl.BlockSpec(memory_space=pl.ANY),
            pl.BlockSpec(memory_space=pl.ANY),
        ],
        out_specs=pl.BlockSpec((TM, H), lambda s, te_r, re_r: (s, 0)),
        scratch_shapes=[
            pltpu.VMEM((2, FF, H), jnp.float32),
            pltpu.VMEM((2, H, FF), jnp.float32),
            pltpu.SMEM((2,), jnp.int32),
            pltpu.SemaphoreType.DMA((2,)),
            pltpu.SemaphoreType.DMA((2,)),
        ],
    )
    return pl.pallas_call(
        _ffn_body,
        grid_spec=grid_spec,
        out_shape=jax.ShapeDtypeStruct((NPAD, H), jnp.float32),
    )(te, rend, xs, ws, w1, w2)


def kernel(x, Wg, W1, W2):
    b, t, d = x.shape
    assert (b * t, d) == (T, H) and W1.shape == (E, FF, H)
    h = x.reshape(T, H)
    pos0, pos1, w0, w1, te, rend = _route(h, Wg)
    p0 = pos0.reshape(T)
    p1 = pos1.reshape(T)
    sc_scatter, sc_combine = _sc_kernels()
    xs, ws = sc_scatter(h, w0, w1, p0, p1)
    outs = _grouped_ffn(xs, ws, W1, W2, te.reshape(NT), rend.reshape(NT))
    y = sc_combine(outs, p0, p1)
    return y.reshape(b, t, d)
